# baseline (speedup 1.0000x reference)
import functools

import jax
import jax.numpy as jnp
from jax import lax
from jax.experimental import pallas as pl
from jax.experimental.pallas import tpu as pltpu
from jax.experimental.pallas import tpu_sc as plsc

B = 16384
N_DENSE = 13
N_SPARSE = 26
VOCAB = 100000
EMB = 16

NC = 2
NS = 16
NW = NC * NS
L = 16

ROWS_PER_TILE = B // NW
G_PER_TILE = ROWS_PER_TILE * N_SPARSE
CHUNK = 128
N_CHUNKS = G_PER_TILE // CHUNK
SUPER = 13
N_SUPER = N_CHUNKS // SUPER
ROWS_PER_SUPER = SUPER * CHUNK // N_SPARSE


def _sc_partial_sums(xs_flat, table_flat):
  mesh = plsc.VectorSubcoreMesh(core_axis_name="c", subcore_axis_name="s")

  @functools.partial(
      pl.kernel,
      out_type=jax.ShapeDtypeStruct((B * EMB,), jnp.float32),
      mesh=mesh,
      scratch_types=[
          pltpu.VMEM((G_PER_TILE,), jnp.float32),
          pltpu.VMEM((N_CHUNKS, CHUNK), jnp.int32),
          pltpu.VMEM((SUPER * CHUNK, EMB), jnp.float32),
          pltpu.VMEM((ROWS_PER_TILE * EMB,), jnp.float32),
          pltpu.SemaphoreType.DMA,
      ],
      compiler_params=pltpu.CompilerParams(use_tc_tiling_on_sc=False),
  )
  def k(xs_hbm, tab_hbm, out_hbm, xb, idxb, gbuf, acc, sem):
    wid = lax.axis_index("s") * NC + lax.axis_index("c")
    base = wid * G_PER_TILE

    pltpu.sync_copy(xs_hbm.at[pl.ds(base, G_PER_TILE)], xb)

    iota = lax.iota(jnp.int32, L)

    def idx_body(kk, _):
      xv = xb[pl.ds(kk * L, L)]
      pos = iota + kk * L
      f = lax.rem(pos, N_SPARSE)
      flat = xv.astype(jnp.int32) + f * VOCAB
      row = kk // (CHUNK // L)
      col = lax.rem(kk, CHUNK // L) * L
      idxb[row, pl.ds(col, L)] = flat
      return 0

    lax.fori_loop(0, G_PER_TILE // L, idx_body, 0)

    def super_body(s, _):
      copies = []
      for j in range(SUPER):
        copies.append(
            pltpu.async_copy(
                tab_hbm.at[idxb.at[s * SUPER + j]],
                gbuf.at[pl.ds(j * CHUNK, CHUNK)],
                sem,
            ))
      for c in copies:
        c.wait()

      def red_body(g, _):
        v = gbuf[g * N_SPARSE, :]
        for r in range(1, N_SPARSE):
          v = v + gbuf[g * N_SPARSE + r, :]
        acc[pl.ds((s * ROWS_PER_SUPER + g) * EMB, EMB)] = v
        return 0

      lax.fori_loop(0, ROWS_PER_SUPER, red_body, 0)
      return 0

    lax.fori_loop(0, N_SUPER, super_body, 0)

    pltpu.sync_copy(acc, out_hbm.at[pl.ds(wid * ROWS_PER_TILE * EMB,
                                          ROWS_PER_TILE * EMB)])

  return k(xs_flat, table_flat)


def _tc_finish_body(x_ref, p_ref, w_ref, o_ref):
  dense = jax.lax.dot_general(
      x_ref[...], w_ref[...],
      dimension_numbers=(((1,), (0,)), ((), ())),
      preferred_element_type=jnp.float32,
  )
  sparse = jnp.sum(p_ref[...], axis=1, keepdims=True)
  o_ref[...] = jax.nn.sigmoid(dense + sparse)


def _tc_finish(x_dense, partial, weight):
  blk = 2048
  grid = B // blk
  return pl.pallas_call(
      _tc_finish_body,
      grid=(grid,),
      in_specs=[
          pl.BlockSpec((blk, N_DENSE), lambda i: (i, 0)),
          pl.BlockSpec((blk, EMB), lambda i: (i, 0)),
          pl.BlockSpec((N_DENSE, 1), lambda i: (0, 0)),
      ],
      out_specs=pl.BlockSpec((blk, 1), lambda i: (i, 0)),
      out_shape=jax.ShapeDtypeStruct((B, 1), jnp.float32),
  )(x_dense, partial, weight)


@jax.jit
def kernel(X, tables, weight):
  xs_flat = X[:, N_DENSE:N_DENSE + N_SPARSE].reshape(-1)
  table_flat = tables.reshape(N_SPARSE * VOCAB, EMB)
  partial = _sc_partial_sums(xs_flat, table_flat).reshape(B, EMB)
  return _tc_finish(X[:, :N_DENSE], partial, weight)


# speedup vs baseline: 1.0157x; 1.0157x over previous
import functools

import jax
import jax.numpy as jnp
from jax import lax
from jax.experimental import pallas as pl
from jax.experimental.pallas import tpu as pltpu
from jax.experimental.pallas import tpu_sc as plsc

B = 16384
N_DENSE = 13
N_SPARSE = 26
VOCAB = 100000
EMB = 16
NCOL = N_DENSE + N_SPARSE

NC = 2
NS = 16
NW = NC * NS
L = 16

ROWS_PER_TILE = B // NW
G_PER_TILE = ROWS_PER_TILE * N_SPARSE
X_PER_TILE = ROWS_PER_TILE * NCOL
CHUNK = 128
N_CHUNKS = G_PER_TILE // CHUNK
SUPER = 13
N_SUPER = N_CHUNKS // SUPER
ROWS_PER_SUPER = SUPER * CHUNK // N_SPARSE
VPHASE = SUPER
STEPS_PER_SUPER = SUPER * CHUNK // (VPHASE * L)

def _sc_partial_sums(x_flat, table_flat):
  mesh = plsc.VectorSubcoreMesh(core_axis_name="c", subcore_axis_name="s")

  @functools.partial(
      pl.kernel,
      out_type=jax.ShapeDtypeStruct((B * EMB,), jnp.float32),
      mesh=mesh,
      scratch_types=[
          pltpu.VMEM((X_PER_TILE,), jnp.float32),
          pltpu.VMEM((N_CHUNKS, CHUNK), jnp.int32),
          pltpu.VMEM((2, SUPER * CHUNK, EMB), jnp.float32),
          pltpu.VMEM((ROWS_PER_TILE * EMB,), jnp.float32),
          pltpu.SemaphoreType.DMA,
          pltpu.SemaphoreType.DMA,
      ],
      compiler_params=pltpu.CompilerParams(
          use_tc_tiling_on_sc=False, needs_layout_passes=False),
  )
  def k(x_hbm, tab_hbm, out_hbm, xb, idxb, gbuf, acc, sem0, sem1):
    wid = lax.axis_index("s") * NC + lax.axis_index("c")

    pltpu.sync_copy(x_hbm.at[pl.ds(wid * X_PER_TILE, X_PER_TILE)], xb)

    iota = lax.iota(jnp.int32, L)
    src_c = []
    voff_c = []
    n26 = jnp.full((L,), N_SPARSE, jnp.int32)
    for j in range(VPHASE):
      pos = iota + j * L
      b_off = lax.div(pos, n26)
      f = lax.rem(pos, n26)
      src_c.append(b_off * NCOL + N_DENSE + f)
      voff_c.append(f * VOCAB)

    def idx_super(s):
      def body(t, _):
        tp = s * STEPS_PER_SUPER + t
        xoff = tp * (VPHASE * L) // N_SPARSE * NCOL
        kk0 = tp * VPHASE
        xoff_v = jnp.full((L,), xoff, jnp.int32)
        for j in range(VPHASE):
          xv = plsc.load_gather(xb, [src_c[j] + xoff_v])
          flat = xv.astype(jnp.int32) + voff_c[j]
          kk = kk0 + j
          row = kk // (CHUNK // L)
          col = (kk % (CHUNK // L)) * L
          idxb[row, pl.ds(col, L)] = flat
        return 0

      lax.fori_loop(0, STEPS_PER_SUPER, body, 0)

    def fire(s, pbuf, sem):
      return [
          pltpu.async_copy(
              tab_hbm.at[idxb.at[s * SUPER + j]],
              gbuf.at[pbuf, pl.ds(j * CHUNK, CHUNK)],
              sem,
          ) for j in range(SUPER)
      ]

    def reduce(s, pbuf):
      def body(g, _):
        base = g * N_SPARSE
        v0 = gbuf[pbuf, base + 0, :]
        v1 = gbuf[pbuf, base + 1, :]
        v2 = gbuf[pbuf, base + 2, :]
        v3 = gbuf[pbuf, base + 3, :]
        for r in range(4, N_SPARSE, 4):
          v0 = v0 + gbuf[pbuf, base + r, :]
          v1 = v1 + gbuf[pbuf, base + r + 1, :]
          if r + 2 < N_SPARSE:
            v2 = v2 + gbuf[pbuf, base + r + 2, :]
            v3 = v3 + gbuf[pbuf, base + r + 3, :]
        acc[pl.ds((s * ROWS_PER_SUPER + g) * EMB, EMB)] = (v0 + v1) + (v2 + v3)
        return 0

      lax.fori_loop(0, ROWS_PER_SUPER, body, 0)

    sems = [sem0, sem1]
    idx_super(0)
    copies = fire(0, 0, sems[0])
    for s in range(N_SUPER):
      if s + 1 < N_SUPER:
        idx_super(s + 1)
        nxt = fire(s + 1, (s + 1) % 2, sems[(s + 1) % 2])
      for c in copies:
        c.wait()
      reduce(s, s % 2)
      if s + 1 < N_SUPER:
        copies = nxt

    pltpu.sync_copy(
        acc,
        out_hbm.at[pl.ds(wid * ROWS_PER_TILE * EMB, ROWS_PER_TILE * EMB)])

  return k(x_flat, table_flat)


def _tc_finish_body(x_ref, p_ref, w_ref, o_ref):
  dense = jax.lax.dot_general(
      x_ref[...][:, :N_DENSE], w_ref[...],
      dimension_numbers=(((1,), (0,)), ((), ())),
      preferred_element_type=jnp.float32,
  )
  sparse = jnp.sum(p_ref[...], axis=1, keepdims=True)
  o_ref[...] = jax.nn.sigmoid(dense + sparse)


def _tc_finish(x, partial, weight):
  blk = 2048
  grid = B // blk
  return pl.pallas_call(
      _tc_finish_body,
      grid=(grid,),
      in_specs=[
          pl.BlockSpec((blk, NCOL), lambda i: (i, 0)),
          pl.BlockSpec((blk, EMB), lambda i: (i, 0)),
          pl.BlockSpec((N_DENSE, 1), lambda i: (0, 0)),
      ],
      out_specs=pl.BlockSpec((blk, 1), lambda i: (i, 0)),
      out_shape=jax.ShapeDtypeStruct((B, 1), jnp.float32),
  )(x, partial, weight)


@jax.jit
def kernel(X, tables, weight):
  x_flat = X.reshape(-1)
  table_flat = tables.reshape(N_SPARSE * VOCAB, EMB)
  partial = _sc_partial_sums(x_flat, table_flat).reshape(B, EMB)
  return _tc_finish(X, partial, weight)


# speedup vs baseline: 9.8763x; 9.7233x over previous
import functools

import jax
import jax.numpy as jnp
from jax import lax
from jax.experimental import pallas as pl
from jax.experimental.pallas import tpu as pltpu
from jax.experimental.pallas import tpu_sc as plsc

B = 16384
N_DENSE = 13
N_SPARSE = 26
VOCAB = 100000
VPAD = 100352
EMB = 16
NCOL = N_DENSE + N_SPARSE

NC = 2
NS = 16
NW = NC * NS
L = 16

ROWS_PER_TILE = B // NW
G_PER_TILE = ROWS_PER_TILE * N_SPARSE
CHUNK = 128
CBLK = ROWS_PER_TILE // CHUNK


def _table_sum(tables_t):
  def body(t_ref, s_ref):
    s_ref[pl.ds(0, VOCAB)] = jnp.sum(t_ref[...], axis=(0, 1))

  return pl.pallas_call(
      body,
      grid=(N_SPARSE,),
      in_specs=[pl.BlockSpec((1, EMB, VOCAB), lambda f: (f, 0, 0))],
      out_specs=pl.BlockSpec((VPAD,), lambda f: (f,)),
      out_shape=jax.ShapeDtypeStruct((N_SPARSE * VPAD,), jnp.float32),
  )(tables_t)


def _extract_idx(x_t):
  def body(x_ref, o_ref):
    x = x_ref[...]
    for f in range(N_SPARSE):
      o_ref[pl.ds(f * ROWS_PER_TILE, ROWS_PER_TILE)] = (
          x[N_DENSE + f].astype(jnp.int32) + f * VPAD)

  return pl.pallas_call(
      body,
      grid=(NW,),
      in_specs=[pl.BlockSpec((NCOL, ROWS_PER_TILE), lambda w: (0, w))],
      out_specs=pl.BlockSpec((G_PER_TILE,), lambda w: (w,)),
      out_shape=jax.ShapeDtypeStruct((B * N_SPARSE,), jnp.int32),
  )(x_t)


def _sc_gather_sum(idx_flat, s_flat):
  mesh = plsc.VectorSubcoreMesh(core_axis_name="c", subcore_axis_name="s")

  @functools.partial(
      pl.kernel,
      out_type=jax.ShapeDtypeStruct((B,), jnp.float32),
      mesh=mesh,
      scratch_types=[
          pltpu.VMEM((N_SPARSE * CBLK, CHUNK), jnp.int32),
          pltpu.VMEM((2, N_SPARSE, CHUNK), jnp.float32),
          pltpu.VMEM((ROWS_PER_TILE,), jnp.float32),
          pltpu.SemaphoreType.DMA,
          pltpu.SemaphoreType.DMA,
          pltpu.SemaphoreType.DMA,
      ],
      compiler_params=pltpu.CompilerParams(
          use_tc_tiling_on_sc=False, needs_layout_passes=False),
  )
  def k(idx_hbm, s_hbm, out_hbm, idxb, gbuf, outb, semi, sem0, sem1):
    wid = lax.axis_index("s") * NC + lax.axis_index("c")

    stage = [
        pltpu.async_copy(
            idx_hbm.at[pl.ds(wid * G_PER_TILE + r * CHUNK, CHUNK)],
            idxb.at[r], semi)
        for r in range(N_SPARSE * CBLK)
    ]
    for c in stage:
      c.wait()

    def fire(cc, p, sem):
      return [
          pltpu.async_copy(
              s_hbm.at[idxb.at[f * CBLK + cc]],
              gbuf.at[p, f],
              sem,
          ) for f in range(N_SPARSE)
      ]

    def reduce(cc, p):
      for g in range(CHUNK // L):
        v0 = gbuf[p, 0, pl.ds(g * L, L)]
        v1 = gbuf[p, 1, pl.ds(g * L, L)]
        v2 = gbuf[p, 2, pl.ds(g * L, L)]
        v3 = gbuf[p, 3, pl.ds(g * L, L)]
        for f in range(4, N_SPARSE, 4):
          v0 = v0 + gbuf[p, f, pl.ds(g * L, L)]
          v1 = v1 + gbuf[p, f + 1, pl.ds(g * L, L)]
          if f + 2 < N_SPARSE:
            v2 = v2 + gbuf[p, f + 2, pl.ds(g * L, L)]
          if f + 3 < N_SPARSE:
            v3 = v3 + gbuf[p, f + 3, pl.ds(g * L, L)]
        outb[pl.ds(cc * CHUNK + g * L, L)] = (v0 + v1) + (v2 + v3)

    sems = [sem0, sem1]
    copies = fire(0, 0, sems[0])
    for cc in range(CBLK):
      if cc + 1 < CBLK:
        nxt = fire(cc + 1, (cc + 1) % 2, sems[(cc + 1) % 2])
      for c in copies:
        c.wait()
      reduce(cc, cc % 2)
      if cc + 1 < CBLK:
        copies = nxt

    pltpu.sync_copy(outb,
                    out_hbm.at[pl.ds(wid * ROWS_PER_TILE, ROWS_PER_TILE)])

  return k(idx_flat, s_flat)


def _tc_finish(x_t, sparse_sum, weight):
  def body(x_ref, p_ref, w_ref, o_ref):
    dense = jax.lax.dot_general(
        w_ref[...], x_ref[...][:N_DENSE, :],
        dimension_numbers=(((0,), (0,)), ((), ())),
        preferred_element_type=jnp.float32,
    )
    o_ref[...] = jax.nn.sigmoid(dense + p_ref[...][None, :])

  return pl.pallas_call(
      body,
      grid=(1,),
      in_specs=[
          pl.BlockSpec((NCOL, B), lambda i: (0, 0)),
          pl.BlockSpec((B,), lambda i: (0,)),
          pl.BlockSpec((N_DENSE, 1), lambda i: (0, 0)),
      ],
      out_specs=pl.BlockSpec((1, B), lambda i: (0, 0)),
      out_shape=jax.ShapeDtypeStruct((1, B), jnp.float32),
  )(x_t, sparse_sum, weight)


@jax.jit
def kernel(X, tables, weight):
  x_t = X.T
  tables_t = jnp.transpose(tables, (0, 2, 1))
  s_flat = _table_sum(tables_t)
  idx_flat = _extract_idx(x_t)
  sparse_sum = _sc_gather_sum(idx_flat, s_flat)
  out_t = _tc_finish(x_t, sparse_sum, weight)
  return out_t.reshape(B, 1)


# speedup vs baseline: 11.2093x; 1.1350x over previous
import functools

import jax
import jax.numpy as jnp
from jax import lax
from jax.experimental import pallas as pl
from jax.experimental.pallas import tpu as pltpu
from jax.experimental.pallas import tpu_sc as plsc

B = 16384
N_DENSE = 13
N_SPARSE = 26
VOCAB = 100000
VPAD = 100352
EMB = 16
NCOL = N_DENSE + N_SPARSE

NC = 2
NS = 16
NW = NC * NS
L = 16

ROWS_PER_TILE = B // NW
G_PER_TILE = ROWS_PER_TILE * N_SPARSE
CHUNK = 128
CBLK = ROWS_PER_TILE // CHUNK


def _table_sum(tables_t):
  def body(t_ref, s_ref):
    s_ref[pl.ds(0, VOCAB)] = jnp.sum(t_ref[...], axis=(0, 1))

  return pl.pallas_call(
      body,
      grid=(N_SPARSE,),
      in_specs=[pl.BlockSpec((1, EMB, VOCAB), lambda f: (f, 0, 0))],
      out_specs=pl.BlockSpec((VPAD,), lambda f: (f,)),
      out_shape=jax.ShapeDtypeStruct((N_SPARSE * VPAD,), jnp.float32),
  )(tables_t)


def _extract_idx(x_t):
  def body(x_ref, o_ref):
    x = x_ref[...]
    for f in range(N_SPARSE):
      row = x[N_DENSE + f].astype(jnp.int32) + f * VPAD
      for w in range(NW):
        o_ref[pl.ds(w * G_PER_TILE + f * ROWS_PER_TILE, ROWS_PER_TILE)] = (
            row[w * ROWS_PER_TILE:(w + 1) * ROWS_PER_TILE])

  return pl.pallas_call(
      body,
      in_specs=[pl.BlockSpec((NCOL, B), lambda: (0, 0))],
      out_specs=pl.BlockSpec((B * N_SPARSE,), lambda: (0,)),
      out_shape=jax.ShapeDtypeStruct((B * N_SPARSE,), jnp.int32),
  )(x_t)


def _sc_gather_sum(idx_flat, s_flat):
  mesh = plsc.VectorSubcoreMesh(core_axis_name="c", subcore_axis_name="s")

  @functools.partial(
      pl.kernel,
      out_type=jax.ShapeDtypeStruct((B,), jnp.float32),
      mesh=mesh,
      scratch_types=[
          pltpu.VMEM((N_SPARSE * CBLK, CHUNK), jnp.int32),
          pltpu.VMEM((2, N_SPARSE, CHUNK), jnp.float32),
          pltpu.VMEM((ROWS_PER_TILE,), jnp.float32),
          pltpu.SemaphoreType.DMA,
          pltpu.SemaphoreType.DMA,
          pltpu.SemaphoreType.DMA,
      ],
      compiler_params=pltpu.CompilerParams(
          use_tc_tiling_on_sc=False, needs_layout_passes=False),
  )
  def k(idx_hbm, s_hbm, out_hbm, idxb, gbuf, outb, semi, sem0, sem1):
    wid = lax.axis_index("s") * NC + lax.axis_index("c")

    stage = [
        pltpu.async_copy(
            idx_hbm.at[pl.ds(wid * G_PER_TILE + r * CHUNK, CHUNK)],
            idxb.at[r], semi)
        for r in range(N_SPARSE * CBLK)
    ]
    for c in stage:
      c.wait()

    def fire(cc, p, sem):
      return [
          pltpu.async_copy(
              s_hbm.at[idxb.at[f * CBLK + cc]],
              gbuf.at[p, f],
              sem,
          ) for f in range(N_SPARSE)
      ]

    def reduce(cc, p):
      for g in range(CHUNK // L):
        v0 = gbuf[p, 0, pl.ds(g * L, L)]
        v1 = gbuf[p, 1, pl.ds(g * L, L)]
        v2 = gbuf[p, 2, pl.ds(g * L, L)]
        v3 = gbuf[p, 3, pl.ds(g * L, L)]
        for f in range(4, N_SPARSE, 4):
          v0 = v0 + gbuf[p, f, pl.ds(g * L, L)]
          v1 = v1 + gbuf[p, f + 1, pl.ds(g * L, L)]
          if f + 2 < N_SPARSE:
            v2 = v2 + gbuf[p, f + 2, pl.ds(g * L, L)]
          if f + 3 < N_SPARSE:
            v3 = v3 + gbuf[p, f + 3, pl.ds(g * L, L)]
        outb[pl.ds(cc * CHUNK + g * L, L)] = (v0 + v1) + (v2 + v3)

    sems = [sem0, sem1]
    copies = fire(0, 0, sems[0])
    for cc in range(CBLK):
      if cc + 1 < CBLK:
        nxt = fire(cc + 1, (cc + 1) % 2, sems[(cc + 1) % 2])
      for c in copies:
        c.wait()
      reduce(cc, cc % 2)
      if cc + 1 < CBLK:
        copies = nxt

    pltpu.sync_copy(outb,
                    out_hbm.at[pl.ds(wid * ROWS_PER_TILE, ROWS_PER_TILE)])

  return k(idx_flat, s_flat)


def _tc_finish(x_t, sparse_sum, weight):
  def body(x_ref, p_ref, w_ref, o_ref):
    dense = jax.lax.dot_general(
        w_ref[...], x_ref[...][:N_DENSE, :],
        dimension_numbers=(((0,), (0,)), ((), ())),
        preferred_element_type=jnp.float32,
    )
    o_ref[...] = jax.nn.sigmoid(dense + p_ref[...][None, :])

  return pl.pallas_call(
      body,
      grid=(1,),
      in_specs=[
          pl.BlockSpec((NCOL, B), lambda i: (0, 0)),
          pl.BlockSpec((B,), lambda i: (0,)),
          pl.BlockSpec((N_DENSE, 1), lambda i: (0, 0)),
      ],
      out_specs=pl.BlockSpec((1, B), lambda i: (0, 0)),
      out_shape=jax.ShapeDtypeStruct((1, B), jnp.float32),
  )(x_t, sparse_sum, weight)


@jax.jit
def kernel(X, tables, weight):
  x_t = X.T
  tables_t = jnp.transpose(tables, (0, 2, 1))
  s_flat = _table_sum(tables_t)
  idx_flat = _extract_idx(x_t)
  sparse_sum = _sc_gather_sum(idx_flat, s_flat)
  out_t = _tc_finish(x_t, sparse_sum, weight)
  return out_t.reshape(B, 1)


# speedup vs baseline: 11.2631x; 1.0048x over previous
import functools

import jax
import jax.numpy as jnp
from jax import lax
from jax.experimental import pallas as pl
from jax.experimental.pallas import tpu as pltpu
from jax.experimental.pallas import tpu_sc as plsc

B = 16384
N_DENSE = 13
N_SPARSE = 26
VOCAB = 100000
VPAD = 100352
EMB = 16
NCOL = N_DENSE + N_SPARSE

NC = 2
NS = 16
NW = NC * NS
L = 16

ROWS_PER_TILE = B // NW
CHUNK = 128
CBLK = ROWS_PER_TILE // CHUNK

PHASES = ((0, 7), (7, 14), (14, 20), (20, 26))


def _table_sum(tables_t, lo, n):
  def body(t_ref, s_ref):
    s_ref[pl.ds(0, VOCAB)] = jnp.sum(t_ref[...], axis=(0, 1))

  return pl.pallas_call(
      body,
      grid=(n,),
      in_specs=[pl.BlockSpec((1, EMB, VOCAB), lambda f: (f + lo, 0, 0))],
      out_specs=pl.BlockSpec((VPAD,), lambda f: (f,)),
      out_shape=jax.ShapeDtypeStruct((n * VPAD,), jnp.float32),
  )(tables_t)


def _extract_idx(x_t):
  def body(x_ref, *o_refs):
    x = x_ref[...]
    for p, (lo, hi) in enumerate(PHASES):
      n = hi - lo
      for fl in range(n):
        row = x[N_DENSE + lo + fl].astype(jnp.int32) + fl * VPAD
        for w in range(NW):
          o_refs[p][pl.ds((w * n + fl) * ROWS_PER_TILE, ROWS_PER_TILE)] = (
              row[w * ROWS_PER_TILE:(w + 1) * ROWS_PER_TILE])

  return pl.pallas_call(
      body,
      in_specs=[pl.BlockSpec((NCOL, B), lambda: (0, 0))],
      out_specs=[
          pl.BlockSpec((B * (hi - lo),), lambda: (0,)) for lo, hi in PHASES
      ],
      out_shape=[
          jax.ShapeDtypeStruct((B * (hi - lo),), jnp.int32)
          for lo, hi in PHASES
      ],
  )(x_t)


def _sc_gather_sum(idx_flat, s_flat, n):
  g_per_tile = ROWS_PER_TILE * n
  mesh = plsc.VectorSubcoreMesh(core_axis_name="c", subcore_axis_name="s")

  @functools.partial(
      pl.kernel,
      out_type=jax.ShapeDtypeStruct((B,), jnp.float32),
      mesh=mesh,
      scratch_types=[
          pltpu.VMEM((n * CBLK, CHUNK), jnp.int32),
          pltpu.VMEM((2, n, CHUNK), jnp.float32),
          pltpu.VMEM((ROWS_PER_TILE,), jnp.float32),
          pltpu.SemaphoreType.DMA,
          pltpu.SemaphoreType.DMA,
          pltpu.SemaphoreType.DMA,
      ],
      compiler_params=pltpu.CompilerParams(
          use_tc_tiling_on_sc=False, needs_layout_passes=False),
  )
  def k(idx_hbm, s_hbm, out_hbm, idxb, gbuf, outb, semi, sem0, sem1):
    wid = lax.axis_index("s") * NC + lax.axis_index("c")

    stage = [
        pltpu.async_copy(
            idx_hbm.at[pl.ds(wid * g_per_tile + r * CHUNK, CHUNK)],
            idxb.at[r], semi)
        for r in range(n * CBLK)
    ]
    for c in stage:
      c.wait()

    def fire(cc, p, sem):
      return [
          pltpu.async_copy(
              s_hbm.at[idxb.at[f * CBLK + cc]],
              gbuf.at[p, f],
              sem,
          ) for f in range(n)
      ]

    def reduce(cc, p):
      for g in range(CHUNK // L):
        vs = [gbuf[p, f, pl.ds(g * L, L)] for f in range(min(4, n))]
        for f in range(4, n, 4):
          for j in range(4):
            if f + j < n:
              vs[j] = vs[j] + gbuf[p, f + j, pl.ds(g * L, L)]
        while len(vs) > 1:
          vs = [vs[i] + vs[i + 1] for i in range(0, len(vs) - 1, 2)] + (
              [vs[-1]] if len(vs) % 2 else [])
        outb[pl.ds(cc * CHUNK + g * L, L)] = vs[0]

    sems = [sem0, sem1]
    copies = fire(0, 0, sems[0])
    for cc in range(CBLK):
      if cc + 1 < CBLK:
        nxt = fire(cc + 1, (cc + 1) % 2, sems[(cc + 1) % 2])
      for c in copies:
        c.wait()
      reduce(cc, cc % 2)
      if cc + 1 < CBLK:
        copies = nxt

    pltpu.sync_copy(outb,
                    out_hbm.at[pl.ds(wid * ROWS_PER_TILE, ROWS_PER_TILE)])

  return k(idx_flat, s_flat)


def _tc_finish(x_t, partials, weight):
  def body(x_ref, w_ref, *refs):
    p_refs, o_ref = refs[:-1], refs[-1]
    dense = jax.lax.dot_general(
        w_ref[...], x_ref[...][:N_DENSE, :],
        dimension_numbers=(((0,), (0,)), ((), ())),
        preferred_element_type=jnp.float32,
    )
    sp = p_refs[0][...]
    for pr in p_refs[1:]:
      sp = sp + pr[...]
    o_ref[...] = jax.nn.sigmoid(dense + sp[None, :])

  return pl.pallas_call(
      body,
      grid=(1,),
      in_specs=[
          pl.BlockSpec((NCOL, B), lambda i: (0, 0)),
          pl.BlockSpec((N_DENSE, 1), lambda i: (0, 0)),
      ] + [pl.BlockSpec((B,), lambda i: (0,)) for _ in partials],
      out_specs=pl.BlockSpec((1, B), lambda i: (0, 0)),
      out_shape=jax.ShapeDtypeStruct((1, B), jnp.float32),
  )(x_t, weight, *partials)


@jax.jit
def kernel(X, tables, weight):
  x_t = X.T
  tables_t = jnp.transpose(tables, (0, 2, 1))
  idx_phases = _extract_idx(x_t)
  partials = []
  for p, (lo, hi) in enumerate(PHASES):
    s_p = _table_sum(tables_t, lo, hi - lo)
    partials.append(_sc_gather_sum(idx_phases[p], s_p, hi - lo))
  out_t = _tc_finish(x_t, partials, weight)
  return out_t.reshape(B, 1)


# speedup vs baseline: 11.3870x; 1.0110x over previous
import functools

import jax
import jax.numpy as jnp
from jax import lax
from jax.experimental import pallas as pl
from jax.experimental.pallas import tpu as pltpu
from jax.experimental.pallas import tpu_sc as plsc

B = 16384
N_DENSE = 13
N_SPARSE = 26
VOCAB = 100000
VPAD = 100352
EMB = 16
NCOL = N_DENSE + N_SPARSE

NC = 2
NS = 16
NW = NC * NS
L = 16

ROWS_PER_TILE = B // NW
X_PER_TILE = ROWS_PER_TILE * NCOL
CHUNK = 128
CBLK = ROWS_PER_TILE // CHUNK
VREG_PER_F = ROWS_PER_TILE // L

PHASES = ((0, 10), (10, 20), (20, 26))


def _table_sum(tables_t, lo, n):
  def body(t_ref, s_ref):
    s_ref[pl.ds(0, VOCAB)] = jnp.sum(t_ref[...], axis=(0, 1))

  return pl.pallas_call(
      body,
      grid=(n,),
      in_specs=[pl.BlockSpec((1, EMB, VOCAB), lambda f: (f + lo, 0, 0))],
      out_specs=pl.BlockSpec((VPAD,), lambda f: (f,)),
      out_shape=jax.ShapeDtypeStruct((n * VPAD,), jnp.float32),
  )(tables_t)


def _tc_dense(x_t, weight):
  def body(x_ref, w_ref, o_ref):
    dense = jax.lax.dot_general(
        w_ref[...], x_ref[...][:N_DENSE, :],
        dimension_numbers=(((0,), (0,)), ((), ())),
        preferred_element_type=jnp.float32,
    )
    o_ref[...] = dense[0]

  return pl.pallas_call(
      body,
      grid=(1,),
      in_specs=[
          pl.BlockSpec((NCOL, B), lambda i: (0, 0)),
          pl.BlockSpec((N_DENSE, 1), lambda i: (0, 0)),
      ],
      out_specs=pl.BlockSpec((B,), lambda i: (0,)),
      out_shape=jax.ShapeDtypeStruct((B,), jnp.float32),
  )(x_t, weight)


def _extract_idx(x_t):
  def body(x_ref, *o_refs):
    x = x_ref[...]
    for p, (lo, hi) in enumerate(PHASES):
      n = hi - lo
      for fl in range(n):
        row = x[N_DENSE + lo + fl].astype(jnp.int32) + fl * VPAD
        for w in range(NW):
          o_refs[p][pl.ds((w * n + fl) * ROWS_PER_TILE, ROWS_PER_TILE)] = (
              row[w * ROWS_PER_TILE:(w + 1) * ROWS_PER_TILE])

  return pl.pallas_call(
      body,
      in_specs=[pl.BlockSpec((NCOL, B), lambda: (0, 0))],
      out_specs=[
          pl.BlockSpec((B * (hi - lo),), lambda: (0,)) for lo, hi in PHASES
      ],
      out_shape=[
          jax.ShapeDtypeStruct((B * (hi - lo),), jnp.int32)
          for lo, hi in PHASES
      ],
  )(x_t)


def _sc_phase(idx_flat, s_p, n, extras=None):
  final = extras is not None
  g_per_tile = ROWS_PER_TILE * n
  mesh = plsc.VectorSubcoreMesh(core_axis_name="c", subcore_axis_name="s")

  scratch = [
      pltpu.VMEM((n * CBLK, CHUNK), jnp.int32),
      pltpu.VMEM((2, n, CHUNK), jnp.float32),
      pltpu.VMEM((ROWS_PER_TILE,), jnp.float32),
      pltpu.SemaphoreType.DMA,
      pltpu.SemaphoreType.DMA,
      pltpu.SemaphoreType.DMA,
  ]
  if final:
    scratch.insert(3, pltpu.VMEM((3, ROWS_PER_TILE), jnp.float32))

  @functools.partial(
      pl.kernel,
      out_type=jax.ShapeDtypeStruct((B,), jnp.float32),
      mesh=mesh,
      scratch_types=scratch,
      compiler_params=pltpu.CompilerParams(
          use_tc_tiling_on_sc=False, needs_layout_passes=False),
  )
  def k(*refs):
    if final:
      (idx_hbm, s_hbm, e0_hbm, e1_hbm, e2_hbm, out_hbm,
       idxb, gbuf, outb, eb, semi, sem0, sem1) = refs
      e_hbms = (e0_hbm, e1_hbm, e2_hbm)
    else:
      (idx_hbm, s_hbm, out_hbm, idxb, gbuf, outb, semi, sem0, sem1) = refs
    wid = lax.axis_index("s") * NC + lax.axis_index("c")
    base_b = wid * ROWS_PER_TILE

    stage = [
        pltpu.async_copy(
            idx_hbm.at[pl.ds(wid * g_per_tile + r * CHUNK, CHUNK)],
            idxb.at[r], semi)
        for r in range(n * CBLK)
    ]
    ecps = []
    if final:
      for i, e in enumerate(e_hbms):
        ecps.append(pltpu.async_copy(
            e.at[pl.ds(base_b, ROWS_PER_TILE)], eb.at[i], semi))
    for c in stage:
      c.wait()

    def fire(cc, p, sem):
      return [
          pltpu.async_copy(
              s_hbm.at[idxb.at[f * CBLK + cc]],
              gbuf.at[p, f],
              sem,
          ) for f in range(n)
      ]

    def reduce(cc, p):
      for g in range(CHUNK // L):
        vs = [gbuf[p, f, pl.ds(g * L, L)] for f in range(min(4, n))]
        for f in range(4, n, 4):
          for j in range(4):
            if f + j < n:
              vs[j] = vs[j] + gbuf[p, f + j, pl.ds(g * L, L)]
        while len(vs) > 1:
          vs = [vs[i] + vs[i + 1] for i in range(0, len(vs) - 1, 2)] + (
              [vs[-1]] if len(vs) % 2 else [])
        outb[pl.ds(cc * CHUNK + g * L, L)] = vs[0]

    sems = [sem0, sem1]
    copies = fire(0, 0, sems[0])
    for cc in range(CBLK):
      if cc + 1 < CBLK:
        nxt = fire(cc + 1, (cc + 1) % 2, sems[(cc + 1) % 2])
      for c in copies:
        c.wait()
      reduce(cc, cc % 2)
      if cc + 1 < CBLK:
        copies = nxt

    if final:
      for c in ecps:
        c.wait()
      one = jnp.full((L,), 1.0, jnp.float32)
      for t in range(ROWS_PER_TILE // L):
        sl = pl.ds(t * L, L)
        z = outb[sl] + eb[0, sl] + eb[1, sl] + eb[2, sl]
        outb[sl] = one / (one + jnp.exp(-z))

    pltpu.sync_copy(outb, out_hbm.at[pl.ds(base_b, ROWS_PER_TILE)])

  args = (idx_flat, s_p) + (tuple(extras) if final else ())
  return k(*args)


@jax.jit
def kernel(X, tables, weight):
  x_t = X.T
  tables_t = jnp.transpose(tables, (0, 2, 1))
  dense = _tc_dense(x_t, weight)
  idx_phases = _extract_idx(x_t)
  partials = []
  for p, (lo, hi) in enumerate(PHASES[:-1]):
    s_p = _table_sum(tables_t, lo, hi - lo)
    partials.append(_sc_phase(idx_phases[p], s_p, hi - lo))
  lo, hi = PHASES[-1]
  s_p = _table_sum(tables_t, lo, hi - lo)
  out = _sc_phase(idx_phases[-1], s_p, hi - lo,
                  extras=(partials[0], partials[1], dense))
  return out.reshape(B, 1)
